# baseline (speedup 1.0000x reference)
import functools

import jax
import jax.numpy as jnp
from jax.experimental import pallas as pl
from jax.experimental.pallas import tpu as pltpu

COARSE = 128
FINAL = 10

KEY_BLOCK = 1024
N_KEYS = 100000
N_PAD = 100352
NEG_LARGE = -3.0e38


def _score_block_kernel(q_ref, w_ref, k_ref, s_ref):
    i = pl.program_id(0)
    processed = jax.lax.dot(q_ref[...], w_ref[...],
                            preferred_element_type=jnp.float32)
    s = jax.lax.dot_general(processed, k_ref[...],
                            (((1,), (1,)), ((), ())),
                            preferred_element_type=jnp.float32)
    col = i * KEY_BLOCK + jax.lax.broadcasted_iota(jnp.int32, s.shape, 1)
    s_ref[...] = jnp.where(col < N_KEYS, s, NEG_LARGE)


def _scores(queries, keys, W):
    nq = queries.shape[0]
    keys_p = jnp.pad(keys, ((0, N_PAD - N_KEYS), (0, 0)))
    grid = N_PAD // KEY_BLOCK
    return pl.pallas_call(
        _score_block_kernel,
        grid=(grid,),
        in_specs=[
            pl.BlockSpec((nq, 64), lambda i: (0, 0)),
            pl.BlockSpec((64, 64), lambda i: (0, 0)),
            pl.BlockSpec((KEY_BLOCK, 64), lambda i: (i, 0)),
        ],
        out_specs=pl.BlockSpec((nq, KEY_BLOCK), lambda i: (0, i)),
        out_shape=jax.ShapeDtypeStruct((nq, N_PAD), jnp.float32),
    )(queries, W, keys_p)


def kernel(queries, keys, W):
    scores = _scores(queries, keys, W)
    coarse_scores, coarse_idx = jax.lax.top_k(scores, COARSE)
    processed = queries @ W
    cand = jnp.take(keys, coarse_idx, axis=0)
    diff = processed[:, None, :] - cand
    d2 = jnp.sum(diff * diff, axis=-1)
    neg_top, pos = jax.lax.top_k(-d2, FINAL)
    final_idx = jnp.take_along_axis(coarse_idx, pos, axis=1)
    final_dist = -neg_top
    return (final_dist, final_idx)


# speedup vs baseline: 3.2588x; 3.2588x over previous
import functools

import jax
import jax.numpy as jnp
from jax.experimental import pallas as pl
from jax.experimental.pallas import tpu as pltpu

COARSE = 128
FINAL = 10

KEY_BLOCK = 1024
N_KEYS = 100000
N_PAD = 100352
NEG_LARGE = -3.0e38


GROUP = 128
N_GROUPS = N_PAD // GROUP
GPB = KEY_BLOCK // GROUP
N_GROUPS_PAD = 896


def _score_block_kernel(q_ref, w_ref, k_ref, s_ref, gm_ref):
    i = pl.program_id(0)
    processed = jax.lax.dot(q_ref[...], w_ref[...],
                            preferred_element_type=jnp.float32)
    s = jax.lax.dot_general(processed, k_ref[...],
                            (((1,), (1,)), ((), ())),
                            preferred_element_type=jnp.float32)
    col = i * KEY_BLOCK + jax.lax.broadcasted_iota(jnp.int32, s.shape, 1)
    s = jnp.where(col < N_KEYS, s, NEG_LARGE)
    s_ref[...] = s
    nq = s.shape[0]
    gm_ref[0] = jnp.max(s.reshape(nq, GPB, GROUP), axis=-1)


def _scores(queries, keys, W):
    nq = queries.shape[0]
    keys_p = jnp.pad(keys, ((0, N_PAD - N_KEYS), (0, 0)))
    grid = N_PAD // KEY_BLOCK
    scores, gm3 = pl.pallas_call(
        _score_block_kernel,
        grid=(grid,),
        in_specs=[
            pl.BlockSpec((nq, 64), lambda i: (0, 0)),
            pl.BlockSpec((64, 64), lambda i: (0, 0)),
            pl.BlockSpec((KEY_BLOCK, 64), lambda i: (i, 0)),
        ],
        out_specs=[
            pl.BlockSpec((nq, KEY_BLOCK), lambda i: (0, i)),
            pl.BlockSpec((1, nq, GPB), lambda i: (i, 0, 0)),
        ],
        out_shape=[
            jax.ShapeDtypeStruct((nq, N_PAD), jnp.float32),
            jax.ShapeDtypeStruct((grid, nq, GPB), jnp.float32),
        ],
    )(queries, W, keys_p)
    gmax = gm3.transpose(1, 0, 2).reshape(nq, N_GROUPS)
    return scores, gmax


def _select_groups_kernel(gm_ref, gsel_ref, t_ref, m_scr):
    m_scr[...] = gm_ref[...]
    giota = jax.lax.broadcasted_iota(jnp.int32, m_scr.shape, 1)
    kiota = jax.lax.broadcasted_iota(jnp.int32, gsel_ref.shape, 1)

    def body(k, _):
        m = m_scr[...]
        best = jnp.max(m, axis=1)
        sel = m == best[:, None]
        pos = jnp.min(jnp.where(sel, giota, N_GROUPS_PAD), axis=1)
        hit = giota == pos[:, None]
        gsel_ref[...] = jnp.where(kiota == k, pos[:, None], gsel_ref[...])
        t_ref[:, 0] = best
        m_scr[...] = jnp.where(hit, NEG_LARGE, m)
        return 0

    jax.lax.fori_loop(0, COARSE, body, 0)


def _select_groups(gmax):
    nq = gmax.shape[0]
    gm_pad = jnp.pad(gmax, ((0, 0), (0, N_GROUPS_PAD - N_GROUPS)),
                     constant_values=NEG_LARGE)
    return pl.pallas_call(
        _select_groups_kernel,
        in_specs=[pl.BlockSpec((nq, N_GROUPS_PAD), lambda: (0, 0))],
        out_specs=[
            pl.BlockSpec((nq, COARSE), lambda: (0, 0)),
            pl.BlockSpec((nq, 1), lambda: (0, 0)),
        ],
        out_shape=[
            jax.ShapeDtypeStruct((nq, COARSE), jnp.int32),
            jax.ShapeDtypeStruct((nq, 1), jnp.float32),
        ],
        scratch_shapes=[pltpu.VMEM((nq, N_GROUPS_PAD), jnp.float32)],
    )(gm_pad)


BIG = 3.0e38
QB = 256


def _refine_kernel(p_ref, cand_ref, cidx_ref, dist_ref, idx_ref):
    p = p_ref[...]
    cand = cand_ref[...]
    diff = p[:, None, :] - cand
    d2 = jnp.sum(diff * diff, axis=-1)
    cidx = cidx_ref[...]
    pos_iota = jax.lax.broadcasted_iota(jnp.int32, d2.shape, 1)
    for k in range(FINAL):
        m = jnp.min(d2, axis=1)
        sel = d2 == m[:, None]
        pos = jnp.min(jnp.where(sel, pos_iota, COARSE), axis=1)
        hit = pos_iota == pos[:, None]
        dist_ref[:, k] = m
        idx_ref[:, k] = jnp.sum(jnp.where(hit, cidx, 0), axis=1)
        d2 = jnp.where(hit, BIG, d2)


def _refine(processed, cand, coarse_idx):
    nq = processed.shape[0]
    return pl.pallas_call(
        _refine_kernel,
        grid=(nq // QB,),
        in_specs=[
            pl.BlockSpec((QB, 64), lambda i: (i, 0)),
            pl.BlockSpec((QB, COARSE, 64), lambda i: (i, 0, 0)),
            pl.BlockSpec((QB, COARSE), lambda i: (i, 0)),
        ],
        out_specs=[
            pl.BlockSpec((QB, FINAL), lambda i: (i, 0)),
            pl.BlockSpec((QB, FINAL), lambda i: (i, 0)),
        ],
        out_shape=[
            jax.ShapeDtypeStruct((nq, FINAL), jnp.float32),
            jax.ShapeDtypeStruct((nq, FINAL), jnp.int32),
        ],
    )(processed, cand, coarse_idx)


def kernel(queries, keys, W):
    nq = queries.shape[0]
    scores, gmax = _scores(queries, keys, W)
    gsel, t = _select_groups(gmax)
    scores3 = scores.reshape(nq, N_GROUPS, GROUP)
    grouped = jnp.take_along_axis(scores3, gsel[:, :, None], axis=1)
    gidx = gsel[:, :, None] * GROUP + jnp.arange(GROUP, dtype=jnp.int32)
    vals = grouped.reshape(nq, COARSE * GROUP)
    gidx = gidx.reshape(nq, COARSE * GROUP)
    coarse_scores, pos = jax.lax.top_k(vals, COARSE)
    coarse_idx = jnp.take_along_axis(gidx, pos, axis=1)
    processed = queries @ W
    cand = jnp.take(keys, coarse_idx, axis=0)
    final_dist, final_idx = _refine(processed, cand, coarse_idx)
    return (final_dist, final_idx)


# speedup vs baseline: 14.5101x; 4.4525x over previous
import functools

import jax
import jax.numpy as jnp
from jax import lax
from jax.experimental import pallas as pl
from jax.experimental.pallas import tpu as pltpu
from jax.experimental.pallas import tpu_sc as plsc

NC, NS, LANES = 2, 16, 16
NW = NC * NS

COARSE = 128
FINAL = 10
KEY_BLOCK = 1024
N_KEYS = 100000
N_PAD = 100352
NEG_LARGE = -3.0e38
BIG = 3.0e38
BIGI = 2**30
GROUP = 128
N_GROUPS = N_PAD // GROUP
GPB = KEY_BLOCK // GROUP
SUB = 16
N_SUB = COARSE * GROUP // SUB
CAND = COARSE * SUB


def _score_block_kernel(q_ref, w_ref, k_ref, s_ref, gm_ref):
    i = pl.program_id(0)
    processed = jax.lax.dot(q_ref[...], w_ref[...],
                            preferred_element_type=jnp.float32)
    s = jax.lax.dot_general(processed, k_ref[...],
                            (((1,), (1,)), ((), ())),
                            preferred_element_type=jnp.float32)
    col = i * KEY_BLOCK + jax.lax.broadcasted_iota(jnp.int32, s.shape, 1)
    s = jnp.where(col < N_KEYS, s, NEG_LARGE)
    s_ref[...] = s
    nq = s.shape[0]
    gm_ref[0] = jnp.max(s.reshape(nq, GPB, GROUP), axis=-1)


def _scores(queries, keys, W):
    nq = queries.shape[0]
    keys_p = jnp.pad(keys, ((0, N_PAD - N_KEYS), (0, 0)))
    grid = N_PAD // KEY_BLOCK
    scores, gm3 = pl.pallas_call(
        _score_block_kernel,
        grid=(grid,),
        in_specs=[
            pl.BlockSpec((nq, 64), lambda i: (0, 0)),
            pl.BlockSpec((64, 64), lambda i: (0, 0)),
            pl.BlockSpec((KEY_BLOCK, 64), lambda i: (i, 0)),
        ],
        out_specs=[
            pl.BlockSpec((nq, KEY_BLOCK), lambda i: (0, i)),
            pl.BlockSpec((1, nq, GPB), lambda i: (i, 0, 0)),
        ],
        out_shape=[
            jax.ShapeDtypeStruct((nq, N_PAD), jnp.float32),
            jax.ShapeDtypeStruct((grid, nq, GPB), jnp.float32),
        ],
    )(queries, W, keys_p)
    gmax = gm3.transpose(1, 0, 2).reshape(nq, N_GROUPS)
    return scores, gmax


def _select_kernel(gm_ref, gsel_ref, m_scr):
    m_scr[...] = gm_ref[...]
    giota = jax.lax.broadcasted_iota(jnp.int32, m_scr.shape, 1)
    kiota = jax.lax.broadcasted_iota(jnp.int32, gsel_ref.shape, 1)
    wpad = m_scr.shape[1]

    def body(k, _):
        m = m_scr[...]
        best = jnp.max(m, axis=1)
        sel = m == best[:, None]
        pos = jnp.min(jnp.where(sel, giota, wpad), axis=1)
        hit = giota == pos[:, None]
        gsel_ref[...] = jnp.where(kiota == k, pos[:, None], gsel_ref[...])
        m_scr[...] = jnp.where(hit, NEG_LARGE, m)
        return 0

    jax.lax.fori_loop(0, COARSE, body, 0)


def _select(m, qb=256):
    nq, w = m.shape
    wpad = ((w + 127) // 128) * 128
    if wpad != w:
        m = jnp.pad(m, ((0, 0), (0, wpad - w)), constant_values=NEG_LARGE)
    return pl.pallas_call(
        _select_kernel,
        grid=(nq // qb,),
        in_specs=[pl.BlockSpec((qb, wpad), lambda i: (i, 0))],
        out_specs=pl.BlockSpec((qb, COARSE), lambda i: (i, 0)),
        out_shape=jax.ShapeDtypeStruct((nq, COARSE), jnp.int32),
        scratch_shapes=[pltpu.VMEM((qb, wpad), jnp.float32)],
    )(m)


def _sc_gather_pairs(vals2d, kidx2d, sel, stride_v, stride_k, width):
    nq = sel.shape[0]
    qpw = nq // NW
    mesh = plsc.VectorSubcoreMesh(core_axis_name="c", subcore_axis_name="s",
                                  num_cores=NC, num_subcores=NS)

    @functools.partial(
        pl.kernel,
        out_type=[
            jax.ShapeDtypeStruct((nq, COARSE, width), jnp.float32),
            jax.ShapeDtypeStruct((nq, COARSE, width), jnp.int32),
        ],
        mesh=mesh,
        scratch_types=[
            pltpu.VMEM((COARSE,), jnp.int32),
            pltpu.VMEM((COARSE,), jnp.int32),
            pltpu.VMEM((COARSE,), jnp.int32),
            pltpu.VMEM((COARSE, width), jnp.float32),
            pltpu.VMEM((COARSE, width), jnp.int32),
            pltpu.SemaphoreType.DMA,
        ],
    )
    def body(v_hbm, k_hbm, sel_hbm, vout, kout,
             sel_v, fv_v, fk_v, vrow, krow, sem):
        wid = lax.axis_index("s") * NC + lax.axis_index("c")

        def per_query(qi, _):
            q = wid * qpw + qi
            pltpu.sync_copy(sel_hbm.at[q], sel_v)
            for v in range(COARSE // LANES):
                g = sel_v[pl.ds(v * LANES, LANES)]
                fv_v[pl.ds(v * LANES, LANES)] = g + q * stride_v
                fk_v[pl.ds(v * LANES, LANES)] = g + q * stride_k
            pltpu.async_copy(v_hbm.at[fv_v], vrow, sem).wait()
            pltpu.async_copy(k_hbm.at[fk_v], krow, sem).wait()
            pltpu.sync_copy(vrow, vout.at[q])
            pltpu.sync_copy(krow, kout.at[q])
            return 0

        lax.fori_loop(0, qpw, per_query, 0)

    return body(vals2d, kidx2d, sel)


CB = 8


def _compact_kernel(g_ref, ki_ref, sel_ref, cv_ref, ck_ref):
    sel = sel_ref[...]
    oh_iota = jax.lax.broadcasted_iota(jnp.int32, (CB, COARSE, N_SUB), 2)
    onehot = (sel[:, :, None] == oh_iota).astype(jnp.float32)
    for b in range(CB):
        cv_ref[b] = jax.lax.dot(onehot[b], g_ref[b],
                                preferred_element_type=jnp.float32)
        ck = jax.lax.dot(onehot[b], ki_ref[b],
                         preferred_element_type=jnp.float32)
        ck_ref[b] = ck.astype(jnp.int32)


def _compact_sub(grouped3, gkidxf3, gsel2):
    nq = grouped3.shape[0]
    return pl.pallas_call(
        _compact_kernel,
        grid=(nq // CB,),
        in_specs=[
            pl.BlockSpec((CB, N_SUB, SUB), lambda i: (i, 0, 0)),
            pl.BlockSpec((CB, N_SUB, SUB), lambda i: (i, 0, 0)),
            pl.BlockSpec((CB, COARSE), lambda i: (i, 0)),
        ],
        out_specs=[
            pl.BlockSpec((CB, COARSE, SUB), lambda i: (i, 0, 0)),
            pl.BlockSpec((CB, COARSE, SUB), lambda i: (i, 0, 0)),
        ],
        out_shape=[
            jax.ShapeDtypeStruct((nq, COARSE, SUB), jnp.float32),
            jax.ShapeDtypeStruct((nq, COARSE, SUB), jnp.int32),
        ],
    )(grouped3, gkidxf3, gsel2)


SB = 32


def _submax_kernel(g_ref, o_ref):
    g = g_ref[...]
    o_ref[...] = jnp.max(g.reshape(g.shape[0], N_SUB, SUB), axis=-1)


def _submax(grouped):
    nq = grouped.shape[0]
    return pl.pallas_call(
        _submax_kernel,
        grid=(nq // SB,),
        in_specs=[pl.BlockSpec((SB, COARSE * GROUP), lambda i: (i, 0))],
        out_specs=pl.BlockSpec((SB, N_SUB), lambda i: (i, 0)),
        out_shape=jax.ShapeDtypeStruct((nq, N_SUB), jnp.float32),
    )(grouped)


def _extract_kernel(v_ref, ki_ref, out_ref, v_scr):
    v_scr[...] = v_ref[...]
    kidx = ki_ref[...]
    kiota = jax.lax.broadcasted_iota(jnp.int32, out_ref.shape, 1)

    def body(k, _):
        v = v_scr[...]
        best = jnp.max(v, axis=1)
        sel = v == best[:, None]
        ci = jnp.min(jnp.where(sel, kidx, BIGI), axis=1)
        out_ref[...] = jnp.where(kiota == k, ci[:, None], out_ref[...])
        v_scr[...] = jnp.where(kidx == ci[:, None], NEG_LARGE, v)
        return 0

    jax.lax.fori_loop(0, COARSE, body, 0)


def _extract_coarse(cvals, ckidx, qb=256):
    nq = cvals.shape[0]
    return pl.pallas_call(
        _extract_kernel,
        grid=(nq // qb,),
        in_specs=[
            pl.BlockSpec((qb, CAND), lambda i: (i, 0)),
            pl.BlockSpec((qb, CAND), lambda i: (i, 0)),
        ],
        out_specs=pl.BlockSpec((qb, COARSE), lambda i: (i, 0)),
        out_shape=jax.ShapeDtypeStruct((nq, COARSE), jnp.int32),
        scratch_shapes=[pltpu.VMEM((qb, CAND), jnp.float32)],
    )(cvals, ckidx)


def _sc_gather_keys(keys128, cidx):
    nq = cidx.shape[0]
    qpw = nq // NW
    mesh = plsc.VectorSubcoreMesh(core_axis_name="c", subcore_axis_name="s",
                                  num_cores=NC, num_subcores=NS)

    @functools.partial(
        pl.kernel,
        out_type=jax.ShapeDtypeStruct((nq, COARSE, 128), jnp.float32),
        mesh=mesh,
        scratch_types=[
            pltpu.VMEM((COARSE,), jnp.int32),
            pltpu.VMEM((COARSE, 128), jnp.float32),
            pltpu.SemaphoreType.DMA,
        ],
    )
    def body(keys_hbm, cidx_hbm, out_hbm, idx_v, rows_v, sem):
        wid = lax.axis_index("s") * NC + lax.axis_index("c")

        def per_query(qi, _):
            q = wid * qpw + qi
            pltpu.sync_copy(cidx_hbm.at[q], idx_v)
            pltpu.async_copy(keys_hbm.at[idx_v], rows_v, sem).wait()
            pltpu.sync_copy(rows_v, out_hbm.at[q])
            return 0

        lax.fori_loop(0, qpw, per_query, 0)

    return body(keys128, cidx)


QB = 128


def _refine_kernel(p_ref, cand_ref, cidx_ref, dist_ref, idx_ref):
    p = p_ref[...]
    cand = cand_ref[...][:, :, :64]
    diff = p[:, None, :] - cand
    d2 = jnp.sum(diff * diff, axis=-1)
    cidx = cidx_ref[...]
    pos_iota = jax.lax.broadcasted_iota(jnp.int32, d2.shape, 1)
    for k in range(FINAL):
        m = jnp.min(d2, axis=1)
        sel = d2 == m[:, None]
        pos = jnp.min(jnp.where(sel, pos_iota, COARSE), axis=1)
        hit = pos_iota == pos[:, None]
        dist_ref[:, k] = m
        idx_ref[:, k] = jnp.sum(jnp.where(hit, cidx, 0), axis=1)
        d2 = jnp.where(hit, BIG, d2)


def _refine(processed, cand, cidx):
    nq = processed.shape[0]
    return pl.pallas_call(
        _refine_kernel,
        grid=(nq // QB,),
        in_specs=[
            pl.BlockSpec((QB, 64), lambda i: (i, 0)),
            pl.BlockSpec((QB, COARSE, 128), lambda i: (i, 0, 0)),
            pl.BlockSpec((QB, COARSE), lambda i: (i, 0)),
        ],
        out_specs=[
            pl.BlockSpec((QB, FINAL), lambda i: (i, 0)),
            pl.BlockSpec((QB, FINAL), lambda i: (i, 0)),
        ],
        out_shape=[
            jax.ShapeDtypeStruct((nq, FINAL), jnp.float32),
            jax.ShapeDtypeStruct((nq, FINAL), jnp.int32),
        ],
    )(processed, cand, cidx)


def kernel(queries, keys, W):
    nq = queries.shape[0]
    scores, gmax = _scores(queries, keys, W)
    gsel = _select(gmax)
    scores2d = scores.reshape(nq * N_GROUPS, GROUP)
    kidxg = jnp.arange(N_GROUPS * GROUP, dtype=jnp.int32).reshape(
        N_GROUPS, GROUP)
    grouped, gkidx = _sc_gather_pairs(scores2d, kidxg, gsel,
                                      N_GROUPS, 0, GROUP)
    if True:
        sub = jnp.max(grouped.reshape(nq, N_SUB, SUB), axis=-1)
        gsel2 = _select(sub)
        cand = jnp.take_along_axis(grouped.reshape(nq, N_SUB, SUB),
                                   gsel2[:, :, None], axis=1)
        ckidx = jnp.take_along_axis(gkidx.reshape(nq, N_SUB, SUB),
                                    gsel2[:, :, None], axis=1)
    else:
        sub = _submax(grouped.reshape(nq, COARSE * GROUP))
        gsel2 = _select(sub)
        gkidxf = gkidx.reshape(nq, N_SUB, SUB).astype(jnp.float32)
        cand, ckidx = _compact_sub(grouped.reshape(nq, N_SUB, SUB),
                                   gkidxf, gsel2)
    coarse_idx = _extract_coarse(cand.reshape(nq, CAND),
                                 ckidx.reshape(nq, CAND))
    keys128 = jnp.pad(keys, ((0, 0), (0, 64)))
    cand_keys = _sc_gather_keys(keys128, coarse_idx)
    processed = queries @ W
    return _refine(processed, cand_keys, coarse_idx)
